# baseline (speedup 1.0000x reference)
import jax
import jax.numpy as jnp
from jax.experimental import pallas as pl
from jax.experimental.pallas import tpu as pltpu

N_OP, N_MA, N_AG = 100000, 50000, 10000
NC = 128
GC = 32
GF = 64
GGC = 128

B = 10000
S_OP = N_OP // B
S_MA = N_MA // B
S_AG = N_AG // B
GRID = S_OP + S_MA + S_AG


def _ln(x, g, b):
    m = jnp.mean(x, axis=-1, keepdims=True)
    v = jnp.mean((x - m) * (x - m), axis=-1, keepdims=True)
    return (x - m) * jax.lax.rsqrt(v + 1e-5) * g + b


def _accum(x_blk, Wl, bl, token, Wr, br, att_col, s_ref, acc_ref):
    xl = jnp.dot(x_blk, Wl, preferred_element_type=jnp.float32)
    xr = jnp.dot(token, Wr, preferred_element_type=jnp.float32) + br
    z = xl + (xr + bl)
    lr = jnp.maximum(z, 0.2 * z)
    e = jnp.dot(lr, att_col, preferred_element_type=jnp.float32)
    p = jnp.exp(e)
    s_ref[0, 0] += jnp.sum(p)
    acc_ref[...] += jax.lax.dot_general(p, x_blk, (((0,), (0,)), ((), ())),
                                        preferred_element_type=jnp.float32)


def _finalize(s_ref, acc_ref, Wl, bl, bias, g, b):
    acc = jnp.dot(acc_ref[...], Wl,
                  preferred_element_type=jnp.float32) / s_ref[0, 0]
    out = acc + bl + bias
    return jnp.tanh(_ln(out, g, b))


def _body(x_op_ref, x_ma_ref, x_ag_ref, ga_ref,
          op_token, op_Wl, op_bl, op_Wr, op_br, op_att, op_bias, op_g, op_b,
          ma_token, ma_Wl, ma_bl, ma_Wr, ma_br, ma_att, ma_bias, ma_g, ma_b,
          ag_token, ag_Wl, ag_bl, ag_Wr, ag_br, ag_att, ag_bias, ag_g, ag_b,
          rl1_W1, rl1_b1, rl1_W2, rl1_b2, rl1_Wp, rl1_bp, rl1_g, rl1_be,
          rl2_W1, rl2_b1, rl2_W2, rl2_b2, rl2_g, rl2_be, Wf, bf,
          f_op_ref, f_ma_ref, f_ag_ref, gf_ref,
          s_op, a_op, s_ma, a_ma, s_ag, a_ag):
    i = pl.program_id(0)

    @pl.when(i == 0)
    def _init():
        for s_r, a_r in ((s_op, a_op), (s_ma, a_ma), (s_ag, a_ag)):
            s_r[0, 0] = 0.0
            a_r[...] = jnp.zeros_like(a_r)

    @pl.when(jnp.logical_and(i % 3 != 2, i < GRID - 1))
    def _op():
        _accum(x_op_ref[...], op_Wl[...], op_bl[...], op_token[...],
               op_Wr[...], op_br[...], op_att[...], s_op, a_op)

    @pl.when(jnp.logical_and(i % 3 == 2, i < GRID - 1))
    def _ma():
        _accum(x_ma_ref[...], ma_Wl[...], ma_bl[...], ma_token[...],
               ma_Wr[...], ma_br[...], ma_att[...], s_ma, a_ma)

    @pl.when(i == GRID - 1)
    def _ag():
        _accum(x_ag_ref[...], ag_Wl[...], ag_bl[...], ag_token[...],
               ag_Wr[...], ag_br[...], ag_att[...], s_ag, a_ag)

    @pl.when(i == GRID - 1)
    def _final():
        f_op = _finalize(s_op, a_op, op_Wl[...], op_bl[...], op_bias[...],
                         op_g[...], op_b[...])
        f_ma = _finalize(s_ma, a_ma, ma_Wl[...], ma_bl[...], ma_bias[...],
                         ma_g[...], ma_b[...])
        f_ag = _finalize(s_ag, a_ag, ag_Wl[...], ag_bl[...], ag_bias[...],
                         ag_g[...], ag_b[...])
        f_op_ref[...] = f_op
        f_ma_ref[...] = f_ma
        f_ag_ref[...] = f_ag
        cat = jnp.concatenate([ga_ref[...], f_op, f_ma, f_ag], axis=1)
        h = jnp.dot(jnp.tanh(jnp.dot(cat, rl1_W1[...],
                                     preferred_element_type=jnp.float32)
                             + rl1_b1[...]),
                    rl1_W2[...], preferred_element_type=jnp.float32) + rl1_b2[...]
        y = jnp.tanh(_ln(jnp.dot(cat, rl1_Wp[...],
                                 preferred_element_type=jnp.float32)
                         + rl1_bp[...] + h, rl1_g[...], rl1_be[...]))
        h2 = jnp.dot(jnp.tanh(jnp.dot(y, rl2_W1[...],
                                      preferred_element_type=jnp.float32)
                              + rl2_b1[...]),
                     rl2_W2[...], preferred_element_type=jnp.float32) + rl2_b2[...]
        y2 = jnp.tanh(_ln(y + h2, rl2_g[...], rl2_be[...]))
        gf_ref[...] = jnp.dot(y2, Wf[...],
                              preferred_element_type=jnp.float32) + bf[...]


def _full(shape):
    nd = len(shape)
    return pl.BlockSpec(shape, lambda i, _n=nd: (0,) * _n)


def kernel(x_operation, x_machine, x_AGV, global_attr, op_token, op_Wl, op_bl, op_Wr, op_br, op_att, op_bias, op_g, op_b, ma_token, ma_Wl, ma_bl, ma_Wr, ma_br, ma_att, ma_bias, ma_g, ma_b, ag_token, ag_Wl, ag_bl, ag_Wr, ag_br, ag_att, ag_bias, ag_g, ag_b, rl1_W1, rl1_b1, rl1_W2, rl1_b2, rl1_Wp, rl1_bp, rl1_g, rl1_be, rl2_W1, rl2_b1, rl2_W2, rl2_b2, rl2_g, rl2_be, Wf, bf):
    row = lambda v: v.reshape(1, -1)
    col = lambda v: v.reshape(-1, 1)

    in_specs = [
        pl.BlockSpec((B, NC),
                     lambda i: (jnp.minimum(i - (i + 1) // 3, S_OP - 1), 0)),
        pl.BlockSpec((B, NC), lambda i: (jnp.clip(i // 3, 0, S_MA - 1), 0)),
        pl.BlockSpec((B, NC), lambda i: (0, 0)),
        _full((1, GF)),
    ]
    small = []
    for tok, Wl, blv, Wr, brv, attv, biasv, gv, bv in (
            (op_token, op_Wl, op_bl, op_Wr, op_br, op_att, op_bias, op_g, op_b),
            (ma_token, ma_Wl, ma_bl, ma_Wr, ma_br, ma_att, ma_bias, ma_g, ma_b),
            (ag_token, ag_Wl, ag_bl, ag_Wr, ag_br, ag_att, ag_bias, ag_g, ag_b)):
        small += [row(tok), Wl, row(blv), Wr, row(brv), col(attv),
                  row(biasv), row(gv), row(bv)]
    small += [rl1_W1, row(rl1_b1), rl1_W2, row(rl1_b2), rl1_Wp, row(rl1_bp),
              row(rl1_g), row(rl1_be),
              rl2_W1, row(rl2_b1), rl2_W2, row(rl2_b2), row(rl2_g), row(rl2_be),
              Wf, row(bf)]
    in_specs += [_full(a.shape) for a in small]

    out_shape = [
        jax.ShapeDtypeStruct((1, GC), jnp.float32),
        jax.ShapeDtypeStruct((1, GC), jnp.float32),
        jax.ShapeDtypeStruct((1, GC), jnp.float32),
        jax.ShapeDtypeStruct((1, GGC), jnp.float32),
    ]
    out_specs = [_full((1, GC)), _full((1, GC)), _full((1, GC)),
                 _full((1, GGC))]

    scratch = []
    for _ in range(3):
        scratch += [pltpu.SMEM((1, 1), jnp.float32),
                    pltpu.VMEM((1, NC), jnp.float32)]

    f_op, f_ma, f_ag, gf = pl.pallas_call(
        _body,
        grid=(GRID,),
        in_specs=in_specs,
        out_specs=out_specs,
        out_shape=out_shape,
        scratch_shapes=scratch,
        compiler_params=pltpu.CompilerParams(
            dimension_semantics=("arbitrary",)),
    )(x_operation, x_machine, x_AGV, row(global_attr), *small)

    return (f_op.reshape(GC), f_ma.reshape(GC), f_ag.reshape(GC),
            gf.reshape(GGC))
